# baseline (speedup 1.0000x reference)
import functools

import jax
import jax.numpy as jnp
from jax import lax
from jax.experimental import pallas as pl
from jax.experimental.pallas import tpu as pltpu
from jax.experimental.pallas import tpu_sc as plsc

B = 16384
K = 16
NC = 2
NS = 16
NW = NC * NS
BPW = B // NW
CH = 128
NCH = BPW // CH
G = BPW // 16
D = 2
NB = D + 1

_mesh = plsc.VectorSubcoreMesh(core_axis_name="c", subcore_axis_name="s")


@functools.partial(
    pl.kernel,
    out_type=jax.ShapeDtypeStruct((B,), jnp.float32),
    mesh=_mesh,
    scratch_types=[
        pltpu.VMEM((NCH, CH), jnp.int32),
        pltpu.VMEM((NCH, CH), jnp.int32),
        pltpu.VMEM((NB * 16 * K, 16), jnp.float32),
        pltpu.VMEM((NB * 16 * K, 16), jnp.float32),
        pltpu.VMEM((BPW,), jnp.float32),
        pltpu.SemaphoreType.DMA,
    ],
    compiler_params=pltpu.CompilerParams(
        needs_layout_passes=False, use_tc_tiling_on_sc=False),
)
def _bpr_sc(uid2d, iid2d, umat_t, imat_t, out, uidx, iidx, ublk, vblk, outv, sem):
    wid = lax.axis_index("s") * NC + lax.axis_index("c")
    pltpu.sync_copy(uid2d.at[pl.ds(wid * NCH, NCH)], uidx)
    pltpu.sync_copy(iid2d.at[pl.ds(wid * NCH, NCH)], iidx)

    lane = lax.iota(jnp.int32, 16)

    def idx_vecs(g):
        r = lax.div(g * 16, CH)
        c = lax.rem(g * 16, CH)
        return uidx[r, pl.ds(c, 16)], iidx[r, pl.ds(c, 16)]

    def fire_group(g, slot):
        uvec, vvec = idx_vecs(g)
        ustart = uvec & -16
        vstart = vvec & -16
        base = slot * 16 * K
        for i in range(16):
            us = pl.multiple_of(ustart[i], 8)
            vs = pl.multiple_of(vstart[i], 8)
            pltpu.async_copy(
                umat_t.at[:, pl.ds(us, 16)],
                ublk.at[pl.ds(base + i * K, K)], sem)
            pltpu.async_copy(
                imat_t.at[:, pl.ds(vs, 16)],
                vblk.at[pl.ds(base + i * K, K)], sem)

    for g in range(D):
        fire_group(g, g)

    def group(g, _):
        for _i in range(16):
            pltpu.make_async_copy(
                umat_t.at[:, pl.ds(0, 16)], ublk.at[pl.ds(0, K)], sem).wait()
            pltpu.make_async_copy(
                imat_t.at[:, pl.ds(0, 16)], vblk.at[pl.ds(0, K)], sem).wait()

        @pl.when(g + D < G)
        def _():
            fire_group(g + D, lax.rem(g + D, NB))

        uvec, vvec = idx_vecs(g)
        ucols = uvec & 15
        vcols = vvec & 15
        rows0 = lax.rem(g, NB) * (16 * K) + lane * K
        acc = jnp.zeros((16,), jnp.float32)
        for k in range(K):
            uc = plsc.load_gather(ublk, [rows0 + k, ucols])
            vc = plsc.load_gather(vblk, [rows0 + k, vcols])
            acc = acc + uc * vc
        outv[pl.ds(g * 16, 16)] = acc
        return 0

    lax.fori_loop(0, G, group, 0)
    pltpu.sync_copy(outv, out.at[pl.ds(wid * BPW, BPW)])


def kernel(uid, iid, user_mat, item_mat):
    uid2d = uid.astype(jnp.int32).reshape((B // CH, CH))
    iid2d = iid.astype(jnp.int32).reshape((B // CH, CH))
    return _bpr_sc(uid2d, iid2d, user_mat.T, item_mat.T)


# speedup vs baseline: 21.6255x; 21.6255x over previous
import functools

import jax
import jax.numpy as jnp
from jax import lax
from jax.experimental import pallas as pl
from jax.experimental.pallas import tpu as pltpu
from jax.experimental.pallas import tpu_sc as plsc

B = 16384
K = 16
NC = 2
NS = 16
NW = NC * NS
BPW = B // NW
CH = 128
NCH = BPW // CH
G = BPW // 16
TBLK = 128

_mesh = plsc.VectorSubcoreMesh(core_axis_name="c", subcore_axis_name="s")


@functools.partial(
    pl.kernel,
    out_type=jax.ShapeDtypeStruct((NW * 8, 128), jnp.float32),
    mesh=_mesh,
    scratch_types=[
        pltpu.VMEM((NCH, CH), jnp.int32),
        pltpu.VMEM((NCH, CH), jnp.int32),
        pltpu.VMEM((16 * K, TBLK), jnp.float32),
        pltpu.VMEM((16 * K, TBLK), jnp.float32),
        pltpu.VMEM((BPW // 8, 128), jnp.float32),
        pltpu.VMEM((BPW // 8, 128), jnp.float32),
        pltpu.VMEM((8, 128), jnp.float32),
        pltpu.SemaphoreType.DMA,
    ],
    compiler_params=pltpu.CompilerParams(
        needs_layout_passes=False, use_tc_tiling_on_sc=True),
)
def _bpr_sc(uid2d, iid2d, umat_t, imat_t, out,
            uidx, iidx, ublk, vblk, urows, vrows, outv, sem):
    wid = lax.axis_index("s") * NC + lax.axis_index("c")
    pltpu.sync_copy(uid2d.at[pl.ds(wid * NCH, NCH)], uidx)
    pltpu.sync_copy(iid2d.at[pl.ds(wid * NCH, NCH)], iidx)

    lane = lax.iota(jnp.int32, 16)

    def idx_vecs(g):
        r = lax.div(g * 16, CH)
        c = lax.rem(g * 16, CH)
        return uidx[r, pl.ds(c, 16)], iidx[r, pl.ds(c, 16)]

    def fire(i, ustart_i, vstart_i):
        us = pl.multiple_of(ustart_i, TBLK)
        vs = pl.multiple_of(vstart_i, TBLK)
        pltpu.async_copy(
            umat_t.at[:, pl.ds(us, TBLK)], ublk.at[pl.ds(i * K, K)], sem)
        pltpu.async_copy(
            imat_t.at[:, pl.ds(vs, TBLK)], vblk.at[pl.ds(i * K, K)], sem)

    uvec0, vvec0 = idx_vecs(0)
    ust0 = (uvec0 >> 7) * TBLK
    vst0 = (vvec0 >> 7) * TBLK
    for i in range(16):
        fire(i, ust0[i], vst0[i])

    def gather_group(g, _):
        uvec, vvec = idx_vecs(g)
        ucol = uvec & (TBLK - 1)
        vcol = vvec & (TBLK - 1)
        has_next = g + 1 < G
        nuvec, nvvec = idx_vecs(lax.rem(g + 1, G))
        nust = (nuvec >> 7) * TBLK
        nvst = (nvvec >> 7) * TBLK
        for i in range(16):
            pltpu.make_async_copy(
                umat_t.at[:, pl.ds(0, TBLK)], ublk.at[pl.ds(0, K)], sem).wait()
            pltpu.make_async_copy(
                imat_t.at[:, pl.ds(0, TBLK)], vblk.at[pl.ds(0, K)], sem).wait()
            uc = plsc.load_gather(
                ublk, [i * K + lane, jnp.full((16,), ucol[i], jnp.int32)])
            vc = plsc.load_gather(
                vblk, [i * K + lane, jnp.full((16,), vcol[i], jnp.int32)])
            urows[2 * g + i // 8, pl.ds((i % 8) * 16, 16)] = uc
            vrows[2 * g + i // 8, pl.ds((i % 8) * 16, 16)] = vc

            @pl.when(has_next)
            def _():
                fire(i, nust[i], nvst[i])

        return 0

    lax.fori_loop(0, G, gather_group, 0)

    def dot_group(g, _):
        rows = 2 * g + (lane >> 3)
        cols0 = (lane & 7) * 16
        acc = jnp.zeros((16,), jnp.float32)
        for k in range(K):
            uc = plsc.load_gather(urows, [rows, cols0 + k])
            vc = plsc.load_gather(vrows, [rows, cols0 + k])
            acc = acc + uc * vc
        outv[lax.div(g, 8), pl.ds(pl.multiple_of(lax.rem(g, 8) * 16, 8), 16)] = acc
        return 0

    lax.fori_loop(0, G, dot_group, 0)
    pltpu.sync_copy(outv, out.at[pl.ds(wid * 8, 8)])


def kernel(uid, iid, user_mat, item_mat):
    uid2d = uid.astype(jnp.int32).reshape((B // CH, CH))
    iid2d = iid.astype(jnp.int32).reshape((B // CH, CH))
    padded = _bpr_sc(uid2d, iid2d, user_mat.T, item_mat.T)
    return padded.reshape(NW, 8, 128)[:, :4, :].reshape(B)
